# baseline (speedup 1.0000x reference)
import functools
import jax
import jax.numpy as jnp
from jax import lax
from jax.experimental import pallas as pl
from jax.experimental.pallas import tpu as pltpu, tpu_sc as plsc

_USER_NUM = 20000
_ITEM_NUM = 30000
_N = _USER_NUM + _ITEM_NUM
_E = 800000
_EMB = 64

_HALF = _N // 2
_NTILES = 16
_NW = 32
_PE = _E // _NW
_PCAP = 13312
_PSTEP = 1000
_PNSTEP = _PE // _PSTEP
_SUB = 128
_EPT = 2 * _PCAP
_NMACRO = _EPT // _SUB
_NBUF = 3
_ACC_ROWS = 25088
_ZROWS = _ACC_ROWS // _NTILES
_CP_ROWS = 1568

_mesh = plsc.VectorSubcoreMesh(core_axis_name="c", subcore_axis_name="s")


@functools.partial(
    pl.kernel,
    mesh=_mesh,
    compiler_params=pltpu.CompilerParams(
        use_tc_tiling_on_sc=False, needs_layout_passes=False),
    out_type=(
        jax.ShapeDtypeStruct((2, _NW, _PCAP), jnp.int32),
        jax.ShapeDtypeStruct((2, _NW, _PCAP), jnp.int32),
        jax.ShapeDtypeStruct((2, _NW, _PCAP), jnp.float32),
    ),
    scratch_types=[
        pltpu.VMEM((_PSTEP + 8,), jnp.int32),
        pltpu.VMEM((_PSTEP + 8,), jnp.int32),
        pltpu.VMEM((_PSTEP + 8,), jnp.float32),
        pltpu.VMEM((_PCAP + 16,), jnp.int32),
        pltpu.VMEM((_PCAP + 16,), jnp.int32),
        pltpu.VMEM((_PCAP + 16,), jnp.float32),
        pltpu.VMEM((_PCAP + 16,), jnp.int32),
        pltpu.VMEM((_PCAP + 16,), jnp.int32),
        pltpu.VMEM((_PCAP + 16,), jnp.float32),
    ],
)
def _partition(row_hbm, col_hbm, val_hbm, oidx, ocol, oval,
               srow, scol, sval, i0, c0, v0, i1, c1, v1):
    c = lax.axis_index("c")
    s = lax.axis_index("s")
    w = c * _NTILES + s
    ebase = w * _PE
    zero16i = jnp.zeros((16,), jnp.int32)
    zero16f = jnp.zeros((16,), jnp.float32)
    lanes = lax.iota(jnp.int32, 16)

    def _zbuf(g, carry):
        fill = (lanes + g * 16) & 8191
        i0[pl.ds(g * 16, 16)] = fill
        c0[pl.ds(g * 16, 16)] = zero16i
        v0[pl.ds(g * 16, 16)] = zero16f
        i1[pl.ds(g * 16, 16)] = fill
        c1[pl.ds(g * 16, 16)] = zero16i
        v1[pl.ds(g * 16, 16)] = zero16f
        return carry

    lax.fori_loop(0, (_PCAP + 16) // 16, _zbuf, 0)

    def _compress(base, tail, ptrs):
        p0, p1 = ptrs
        r = srow[pl.ds(base, 16)]
        cc = scol[pl.ds(base, 16)]
        vv = sval[pl.ds(base, 16)]
        in0 = r < _HALF
        if tail:
            valid = lanes < (_PSTEP % 16)
            keep0 = in0 & valid
            keep1 = jnp.logical_not(in0) & valid
        else:
            keep0 = in0
            keep1 = jnp.logical_not(in0)
        key0 = jnp.where(keep0, lanes, lanes + 16)
        key1 = jnp.where(keep1, lanes, lanes + 16)
        zi = jnp.zeros((16,), jnp.int32)
        zf = jnp.zeros((16,), jnp.float32)
        _, sr0 = plsc.sort_key_val(key0, jnp.where(keep0, r, zi))
        _, sc0 = plsc.sort_key_val(key0, jnp.where(keep0, cc, zi))
        _, sv0 = plsc.sort_key_val(key0, jnp.where(keep0, vv, zf))
        _, sr1 = plsc.sort_key_val(key1, jnp.where(keep1, r - _HALF, zi))
        _, sc1 = plsc.sort_key_val(key1, jnp.where(keep1, cc, zi))
        _, sv1 = plsc.sort_key_val(key1, jnp.where(keep1, vv, zf))
        i0[pl.ds(p0, 16)] = sr0
        c0[pl.ds(p0, 16)] = sc0
        v0[pl.ds(p0, 16)] = sv0
        i1[pl.ds(p1, 16)] = sr1
        c1[pl.ds(p1, 16)] = sc1
        v1[pl.ds(p1, 16)] = sv1
        n0 = plsc.all_reduce_population_count(keep0)[0]
        n1 = plsc.all_reduce_population_count(keep1)[0]
        p0 = jnp.minimum(p0 + n0, _PCAP - 16)
        p1 = jnp.minimum(p1 + n1, _PCAP - 16)
        return (p0, p1)

    full = jnp.ones((16,), jnp.bool_)

    def _step(t, ptrs):
        off = ebase + t * _PSTEP
        pltpu.sync_copy(row_hbm.at[pl.ds(off, _PSTEP)], srow.at[pl.ds(0, _PSTEP)])
        pltpu.sync_copy(col_hbm.at[pl.ds(off, _PSTEP)], scol.at[pl.ds(0, _PSTEP)])
        pltpu.sync_copy(val_hbm.at[pl.ds(off, _PSTEP)], sval.at[pl.ds(0, _PSTEP)])

        def _grp(g, ptrs):
            return _compress(g * 16, False, ptrs)

        ptrs = lax.fori_loop(0, _PSTEP // 16, _grp, ptrs)
        return _compress((_PSTEP // 16) * 16, True, ptrs)

    lax.fori_loop(0, _PNSTEP, _step, (jnp.int32(0), jnp.int32(0)))

    pltpu.sync_copy(i0.at[pl.ds(0, _PCAP)], oidx.at[0, w])
    pltpu.sync_copy(c0.at[pl.ds(0, _PCAP)], ocol.at[0, w])
    pltpu.sync_copy(v0.at[pl.ds(0, _PCAP)], oval.at[0, w])
    pltpu.sync_copy(i1.at[pl.ds(0, _PCAP)], oidx.at[1, w])
    pltpu.sync_copy(c1.at[pl.ds(0, _PCAP)], ocol.at[1, w])
    pltpu.sync_copy(v1.at[pl.ds(0, _PCAP)], oval.at[1, w])


_scratch = []
for _ in range(_NBUF):
    _scratch += [
        pltpu.VMEM((_SUB,), jnp.int32),
        pltpu.VMEM((_SUB,), jnp.int32),
        pltpu.VMEM((_SUB,), jnp.float32),
        pltpu.VMEM((_SUB, _EMB), jnp.float32),
        pltpu.SemaphoreType.DMA,
        pltpu.SemaphoreType.DMA,
    ]
_scratch.append(pltpu.VMEM_SHARED((_ACC_ROWS, _EMB), jnp.float32))


@functools.partial(
    pl.kernel,
    mesh=_mesh,
    compiler_params=pltpu.CompilerParams(use_tc_tiling_on_sc=False),
    out_type=jax.ShapeDtypeStruct((_N, _EMB), jnp.float32),
    scratch_types=_scratch,
)
def _spmm(x_hbm, idx_hbm, col_hbm, val_hbm, y_hbm, *rest):
    bufs = []
    for b in range(_NBUF):
        bufs.append(rest[b * 6:(b + 1) * 6])
    acc = rest[_NBUF * 6]

    c = lax.axis_index("c")
    s = lax.axis_index("s")
    lo = c * _HALF
    zero16 = jnp.zeros((16,), jnp.float32)
    ebase = (c * _NW + 2 * s) * _PCAP

    rowsb0 = bufs[0][3]

    def _zero_rowsb(r, carry):
        for q in range(_EMB // 16):
            rowsb0[r, pl.ds(q * 16, 16)] = zero16
        return carry

    lax.fori_loop(0, _SUB, _zero_rowsb, 0)
    zbase = s * _ZROWS
    for k in range(_ZROWS // _SUB):
        pltpu.sync_copy(rowsb0.at[pl.ds(0, _SUB)],
                        acc.at[pl.ds(zbase + k * _SUB, _SUB)])
    pltpu.sync_copy(rowsb0.at[pl.ds(0, _SUB)],
                    acc.at[pl.ds(zbase + _ZROWS - _SUB, _SUB)])
    plsc.subcore_barrier()

    def _load(m, b):
        colb, idxb, valb = bufs[b][0], bufs[b][1], bufs[b][2]
        off = ebase + m * _SUB
        pltpu.sync_copy(col_hbm.at[pl.ds(off, _SUB)], colb)
        pltpu.sync_copy(idx_hbm.at[pl.ds(off, _SUB)], idxb)
        pltpu.sync_copy(val_hbm.at[pl.ds(off, _SUB)], valb)

    def _gather_start(b):
        colb, rowsb, gsem = bufs[b][0], bufs[b][3], bufs[b][4]
        pltpu.async_copy(x_hbm.at[colb.at[pl.ds(0, _SUB)]], rowsb, gsem)

    def _gather_wait(b):
        colb, rowsb, gsem = bufs[b][0], bufs[b][3], bufs[b][4]
        pltpu.make_async_copy(x_hbm.at[colb.at[pl.ds(0, _SUB)]], rowsb, gsem).wait()

    def _scale(b):
        valb, rowsb = bufs[b][2], bufs[b][3]

        def _grp(g, carry):
            vv = valb[pl.ds(g * 16, 16)]
            for i in range(16):
                e = g * 16 + i
                v = vv[i]
                for q in range(_EMB // 16):
                    rowsb[e, pl.ds(q * 16, 16)] = rowsb[e, pl.ds(q * 16, 16)] * v
            return carry

        lax.fori_loop(0, _SUB // 16, _grp, 0)

    def _scatter_start(b):
        idxb, rowsb, ssem = bufs[b][1], bufs[b][3], bufs[b][5]
        pltpu.async_copy(rowsb, acc.at[idxb], ssem, add=True)

    def _scatter_wait(b):
        idxb, rowsb, ssem = bufs[b][1], bufs[b][3], bufs[b][5]
        pltpu.make_async_copy(rowsb, acc.at[idxb], ssem).wait()

    _load(jnp.int32(0), 0)
    _gather_start(0)

    def _body(k, carry):
        for i in range(_NBUF):
            b = i
            nb = (i + 1) % _NBUF
            m = k * _NBUF + i
            if i == _NBUF - 1:
                _scatter_wait(nb)
            else:
                @pl.when(k > 0)
                def _():
                    _scatter_wait(nb)
            _load(m + 1, nb)
            _gather_start(nb)
            _gather_wait(b)
            _scale(b)
            _scatter_start(b)
        return carry

    _NLOOP = (_NMACRO - 1) // _NBUF
    lax.fori_loop(0, _NLOOP, _body, 0)

    _gather_wait(0)
    _scale(0)
    _scatter_start(0)

    _scatter_wait(1)
    _scatter_wait(2)
    _scatter_wait(0)

    plsc.subcore_barrier()

    start = jnp.minimum(s * _CP_ROWS, _HALF - _CP_ROWS)
    pltpu.sync_copy(acc.at[pl.ds(start, _CP_ROWS)],
                    y_hbm.at[pl.ds(lo + start, _CP_ROWS)])


def _mean_body(a_ref, b_ref, c_ref, o_ref):
    o_ref[...] = (a_ref[...] + b_ref[...] + c_ref[...]) * (1.0 / 3.0)


_R2 = _N * _EMB // 128
_BLK = 1000
_mean3 = pl.pallas_call(
    _mean_body,
    grid=(_R2 // _BLK,),
    in_specs=[pl.BlockSpec((_BLK, 128), lambda i: (i, 0))] * 3,
    out_specs=pl.BlockSpec((_BLK, 128), lambda i: (i, 0)),
    out_shape=jax.ShapeDtypeStruct((_R2, 128), jnp.float32),
)


def kernel(user_emb, item_emb, adj_val, adj_row, adj_col):
    x0 = jnp.concatenate([user_emb, item_emb], axis=0)
    pidx, pcol, pval = _partition(adj_row, adj_col, adj_val)
    pidx = pidx.reshape(-1)
    pcol = pcol.reshape(-1)
    pval = pval.reshape(-1)
    y1 = _spmm(x0, pidx, pcol, pval)
    y2 = _spmm(y1, pidx, pcol, pval)
    y3 = _spmm(y2, pidx, pcol, pval)
    m = _mean3(y1.reshape(_R2, 128), y2.reshape(_R2, 128), y3.reshape(_R2, 128))
    m = m.reshape(_N, _EMB)
    return m[:_USER_NUM], m[_USER_NUM:]


# speedup vs baseline: 1.9179x; 1.9179x over previous
import functools
import jax
import jax.numpy as jnp
from jax import lax
from jax.experimental import pallas as pl
from jax.experimental.pallas import tpu as pltpu, tpu_sc as plsc

_USER_NUM = 20000
_ITEM_NUM = 30000
_N = _USER_NUM + _ITEM_NUM
_E = 800000
_EMB = 64

_HALF = _N // 2
_NTILES = 16
_SUB = 128
_MACRO = 128
_NSUB = _MACRO // _SUB
_EDGES_PER_TILE = _E // _NTILES
_NMACRO = -(-_EDGES_PER_TILE // _MACRO)
_LAST = _EDGES_PER_TILE - (_NMACRO - 1) * _MACRO
_NBUF = 3
_ACC_ROWS = 25088
_ZROWS = _ACC_ROWS // _NTILES
_CP_ROWS = 1568

_mesh = plsc.VectorSubcoreMesh(core_axis_name="c", subcore_axis_name="s")

_scratch = []
for _ in range(_NBUF):
    _scratch += [
        pltpu.VMEM((_MACRO,), jnp.int32),
        pltpu.VMEM((_MACRO,), jnp.int32),
        pltpu.VMEM((_MACRO,), jnp.float32),
        pltpu.VMEM((_NSUB, _SUB), jnp.int32),
        pltpu.VMEM((_MACRO, _EMB), jnp.float32),
        pltpu.SemaphoreType.DMA,
        pltpu.SemaphoreType.DMA,
    ]
_scratch.append(pltpu.VMEM_SHARED((_ACC_ROWS, _EMB), jnp.float32))


@functools.partial(
    pl.kernel,
    mesh=_mesh,
    compiler_params=pltpu.CompilerParams(use_tc_tiling_on_sc=False),
    out_type=jax.ShapeDtypeStruct((_N, _EMB), jnp.float32),
    scratch_types=_scratch,
)
def _spmm(x_hbm, row_hbm, col_hbm, val_hbm, y_hbm, *rest):
    bufs = []
    for b in range(_NBUF):
        bufs.append(rest[b * 7:(b + 1) * 7])
    acc = rest[_NBUF * 7]

    c = lax.axis_index("c")
    s = lax.axis_index("s")
    lo = c * _HALF
    dump = _HALF + s
    zero16 = jnp.zeros((16,), jnp.float32)
    ebase = s * _EDGES_PER_TILE

    rowsb0 = bufs[0][4]

    def _zero_rowsb(r, carry):
        for q in range(_EMB // 16):
            rowsb0[r, pl.ds(q * 16, 16)] = zero16
        return carry

    lax.fori_loop(0, _SUB, _zero_rowsb, 0)
    zbase = s * _ZROWS
    for k in range(_ZROWS // _SUB):
        pltpu.sync_copy(rowsb0.at[pl.ds(0, _SUB)],
                        acc.at[pl.ds(zbase + k * _SUB, _SUB)])
    pltpu.sync_copy(rowsb0.at[pl.ds(0, _SUB)],
                    acc.at[pl.ds(zbase + _ZROWS - _SUB, _SUB)])
    plsc.subcore_barrier()

    def _load(m, b):
        colb, rowb, valb = bufs[b][0], bufs[b][1], bufs[b][2]
        off = ebase + m * _MACRO
        pltpu.sync_copy(col_hbm.at[pl.ds(off, _MACRO)], colb)
        pltpu.sync_copy(row_hbm.at[pl.ds(off, _MACRO)], rowb)
        pltpu.sync_copy(val_hbm.at[pl.ds(off, _MACRO)], valb)

    def _load_last(b):
        colb, rowb, valb = bufs[b][0], bufs[b][1], bufs[b][2]
        off = ebase + (_NMACRO - 1) * _MACRO
        pltpu.sync_copy(col_hbm.at[pl.ds(off, _LAST)], colb.at[pl.ds(0, _LAST)])
        pltpu.sync_copy(row_hbm.at[pl.ds(off, _LAST)], rowb.at[pl.ds(0, _LAST)])
        pltpu.sync_copy(val_hbm.at[pl.ds(off, _LAST)], valb.at[pl.ds(0, _LAST)])
        for t in range(_LAST // 16, _MACRO // 16):
            valb[pl.ds(t * 16, 16)] = zero16

    def _gather_start(b):
        colb, rowsb, gsem = bufs[b][0], bufs[b][4], bufs[b][5]
        for j in range(_NSUB):
            pltpu.async_copy(x_hbm.at[colb.at[pl.ds(j * _SUB, _SUB)]],
                             rowsb.at[pl.ds(j * _SUB, _SUB)], gsem)

    def _gather_wait(b):
        colb, rowsb, gsem = bufs[b][0], bufs[b][4], bufs[b][5]
        for j in range(_NSUB):
            pltpu.make_async_copy(x_hbm.at[colb.at[pl.ds(j * _SUB, _SUB)]],
                                  rowsb.at[pl.ds(j * _SUB, _SUB)], gsem).wait()

    def _index(b):
        rowb, idxb = bufs[b][1], bufs[b][3]
        for j in range(_NSUB):
            for g in range(_SUB // 16):
                r = rowb[pl.ds(j * _SUB + g * 16, 16)]
                local = r - lo
                ok = (local >= 0) & (local < _HALF)
                idxb[j, pl.ds(g * 16, 16)] = jnp.where(ok, local, dump)

    def _scale(b):
        valb, rowsb = bufs[b][2], bufs[b][4]

        def _grp(g, carry):
            vv = valb[pl.ds(g * 16, 16)]
            for i in range(16):
                e = g * 16 + i
                v = vv[i]
                for q in range(_EMB // 16):
                    rowsb[e, pl.ds(q * 16, 16)] = rowsb[e, pl.ds(q * 16, 16)] * v
            return carry

        lax.fori_loop(0, _MACRO // 16, _grp, 0)

    def _scatter_start(b):
        idxb, rowsb, ssem = bufs[b][3], bufs[b][4], bufs[b][6]
        for j in range(_NSUB):
            pltpu.async_copy(rowsb.at[pl.ds(j * _SUB, _SUB)],
                             acc.at[idxb.at[j]], ssem, add=True)

    def _scatter_wait(b):
        idxb, rowsb, ssem = bufs[b][3], bufs[b][4], bufs[b][6]
        for j in range(_NSUB):
            pltpu.make_async_copy(rowsb.at[pl.ds(j * _SUB, _SUB)],
                                  acc.at[idxb.at[j]], ssem).wait()

    _load(jnp.int32(0), 0)
    _gather_start(0)

    def _body(k, carry):
        for i in range(_NBUF):
            b = i
            nb = (i + 1) % _NBUF
            m = k * _NBUF + i
            if i == _NBUF - 1:
                _scatter_wait(nb)
            else:
                @pl.when(k > 0)
                def _():
                    _scatter_wait(nb)
            _load(m + 1, nb)
            _gather_start(nb)
            _index(b)
            _gather_wait(b)
            _scale(b)
            _scatter_start(b)
        return carry

    _NLOOP = (_NMACRO - 4) // _NBUF
    lax.fori_loop(0, _NLOOP, _body, 0)

    def _step(m, b, nb, last=False):
        _scatter_wait(nb)
        if last:
            _load_last(nb)
        else:
            _load(m + 1, nb)
        _gather_start(nb)
        _index(b)
        _gather_wait(b)
        _scale(b)
        _scatter_start(b)

    m0 = _NLOOP * _NBUF
    _step(m0, 0, 1)
    _step(m0 + 1, 1, 2)
    _step(m0 + 2, 2, 0, last=True)
    _index(0)
    _gather_wait(0)
    _scale(0)
    _scatter_start(0)

    _scatter_wait(1)
    _scatter_wait(2)
    _scatter_wait(0)

    plsc.subcore_barrier()

    start = jnp.minimum(s * _CP_ROWS, _HALF - _CP_ROWS)
    pltpu.sync_copy(acc.at[pl.ds(start, _CP_ROWS)],
                    y_hbm.at[pl.ds(lo + start, _CP_ROWS)])


def _mean_body(a_ref, b_ref, c_ref, o_ref):
    o_ref[...] = (a_ref[...] + b_ref[...] + c_ref[...]) * (1.0 / 3.0)


_R2 = _N * _EMB // 128
_BLK = 1000
_mean3 = pl.pallas_call(
    _mean_body,
    grid=(_R2 // _BLK,),
    in_specs=[pl.BlockSpec((_BLK, 128), lambda i: (i, 0))] * 3,
    out_specs=pl.BlockSpec((_BLK, 128), lambda i: (i, 0)),
    out_shape=jax.ShapeDtypeStruct((_R2, 128), jnp.float32),
)


def kernel(user_emb, item_emb, adj_val, adj_row, adj_col):
    x0 = jnp.concatenate([user_emb, item_emb], axis=0)
    y1 = _spmm(x0, adj_row, adj_col, adj_val)
    y2 = _spmm(y1, adj_row, adj_col, adj_val)
    y3 = _spmm(y2, adj_row, adj_col, adj_val)
    m = _mean3(y1.reshape(_R2, 128), y2.reshape(_R2, 128), y3.reshape(_R2, 128))
    m = m.reshape(_N, _EMB)
    return m[:_USER_NUM], m[_USER_NUM:]


# speedup vs baseline: 4.2537x; 2.2179x over previous
import functools
import jax
import jax.numpy as jnp
from jax import lax
from jax.experimental import pallas as pl
from jax.experimental.pallas import tpu as pltpu, tpu_sc as plsc

_USER_NUM = 20000
_ITEM_NUM = 30000
_N = _USER_NUM + _ITEM_NUM
_E = 800000
_EMB = 64

_HALF = _N // 2
_NTILES = 16
_SUB = 128
_MACRO = 128
_NSUB = _MACRO // _SUB
_EDGES_PER_TILE = _E // _NTILES
_NMACRO = -(-_EDGES_PER_TILE // _MACRO)
_LAST = _EDGES_PER_TILE - (_NMACRO - 1) * _MACRO
_NBUF = 3
_ACC_ROWS = 25088
_ZROWS = _ACC_ROWS // _NTILES
_CP_ROWS = 1568

_mesh = plsc.VectorSubcoreMesh(core_axis_name="c", subcore_axis_name="s")

_scratch = []
for _ in range(_NBUF):
    _scratch += [
        pltpu.VMEM((_MACRO,), jnp.int32),
        pltpu.VMEM((_MACRO,), jnp.int32),
        pltpu.VMEM((_MACRO,), jnp.float32),
        pltpu.VMEM((_NSUB, _SUB), jnp.int32),
        pltpu.VMEM((_MACRO, _EMB), jnp.float32),
        pltpu.SemaphoreType.DMA,
        pltpu.SemaphoreType.DMA,
        pltpu.SemaphoreType.DMA,
    ]
_scratch.append(pltpu.VMEM_SHARED((_ACC_ROWS, _EMB), jnp.float32))


@functools.partial(
    pl.kernel,
    mesh=_mesh,
    compiler_params=pltpu.CompilerParams(use_tc_tiling_on_sc=False),
    out_type=jax.ShapeDtypeStruct((_N, _EMB), jnp.float32),
    scratch_types=_scratch,
)
def _spmm(x_hbm, row_hbm, col_hbm, val_hbm, y_hbm, *rest):
    bufs = []
    for b in range(_NBUF):
        bufs.append(rest[b * 8:(b + 1) * 8])
    acc = rest[_NBUF * 8]

    c = lax.axis_index("c")
    s = lax.axis_index("s")
    lo = c * _HALF
    dump = _HALF + s
    zero16 = jnp.zeros((16,), jnp.float32)
    ebase = s * _EDGES_PER_TILE

    rowsb0 = bufs[0][4]

    def _zero_rowsb(r, carry):
        for q in range(_EMB // 16):
            rowsb0[r, pl.ds(q * 16, 16)] = zero16
        return carry

    lax.fori_loop(0, _SUB, _zero_rowsb, 0)
    zbase = s * _ZROWS
    for k in range(_ZROWS // _SUB):
        pltpu.sync_copy(rowsb0.at[pl.ds(0, _SUB)],
                        acc.at[pl.ds(zbase + k * _SUB, _SUB)])
    pltpu.sync_copy(rowsb0.at[pl.ds(0, _SUB)],
                    acc.at[pl.ds(zbase + _ZROWS - _SUB, _SUB)])
    plsc.subcore_barrier()

    def _load_start(m, b):
        colb, rowb, valb, lsem = bufs[b][0], bufs[b][1], bufs[b][2], bufs[b][7]
        off = ebase + m * _MACRO
        pltpu.async_copy(col_hbm.at[pl.ds(off, _MACRO)], colb, lsem)
        pltpu.async_copy(row_hbm.at[pl.ds(off, _MACRO)], rowb, lsem)
        pltpu.async_copy(val_hbm.at[pl.ds(off, _MACRO)], valb, lsem)

    def _load_wait(m, b):
        colb, rowb, valb, lsem = bufs[b][0], bufs[b][1], bufs[b][2], bufs[b][7]
        off = ebase + m * _MACRO
        pltpu.make_async_copy(col_hbm.at[pl.ds(off, _MACRO)], colb, lsem).wait()
        pltpu.make_async_copy(row_hbm.at[pl.ds(off, _MACRO)], rowb, lsem).wait()
        pltpu.make_async_copy(val_hbm.at[pl.ds(off, _MACRO)], valb, lsem).wait()

    def _load_last_start(b):
        colb, rowb, valb, lsem = bufs[b][0], bufs[b][1], bufs[b][2], bufs[b][7]
        off = ebase + (_NMACRO - 1) * _MACRO
        pltpu.async_copy(col_hbm.at[pl.ds(off, _LAST)], colb.at[pl.ds(0, _LAST)], lsem)
        pltpu.async_copy(row_hbm.at[pl.ds(off, _LAST)], rowb.at[pl.ds(0, _LAST)], lsem)
        pltpu.async_copy(val_hbm.at[pl.ds(off, _LAST)], valb.at[pl.ds(0, _LAST)], lsem)

    def _load_last_wait(b):
        colb, rowb, valb, lsem = bufs[b][0], bufs[b][1], bufs[b][2], bufs[b][7]
        off = ebase + (_NMACRO - 1) * _MACRO
        pltpu.make_async_copy(col_hbm.at[pl.ds(off, _LAST)], colb.at[pl.ds(0, _LAST)], lsem).wait()
        pltpu.make_async_copy(row_hbm.at[pl.ds(off, _LAST)], rowb.at[pl.ds(0, _LAST)], lsem).wait()
        pltpu.make_async_copy(val_hbm.at[pl.ds(off, _LAST)], valb.at[pl.ds(0, _LAST)], lsem).wait()
        for t in range(_LAST // 16, _MACRO // 16):
            valb[pl.ds(t * 16, 16)] = zero16

    def _gather_start(b):
        colb, rowsb, gsem = bufs[b][0], bufs[b][4], bufs[b][5]
        for j in range(_NSUB):
            pltpu.async_copy(x_hbm.at[colb.at[pl.ds(j * _SUB, _SUB)]],
                             rowsb.at[pl.ds(j * _SUB, _SUB)], gsem)

    def _gather_wait(b):
        colb, rowsb, gsem = bufs[b][0], bufs[b][4], bufs[b][5]
        for j in range(_NSUB):
            pltpu.make_async_copy(x_hbm.at[colb.at[pl.ds(j * _SUB, _SUB)]],
                                  rowsb.at[pl.ds(j * _SUB, _SUB)], gsem).wait()

    def _index(b):
        rowb, idxb = bufs[b][1], bufs[b][3]
        for j in range(_NSUB):
            for g in range(_SUB // 16):
                r = rowb[pl.ds(j * _SUB + g * 16, 16)]
                local = r - lo
                ok = (local >= 0) & (local < _HALF)
                idxb[j, pl.ds(g * 16, 16)] = jnp.where(ok, local, dump)

    def _scale(b):
        valb, rowsb = bufs[b][2], bufs[b][4]

        def _grp(g, carry):
            vv = valb[pl.ds(g * 16, 16)]
            for i in range(16):
                e = g * 16 + i
                v = vv[i]
                for q in range(_EMB // 16):
                    rowsb[e, pl.ds(q * 16, 16)] = rowsb[e, pl.ds(q * 16, 16)] * v
            return carry

        lax.fori_loop(0, _MACRO // 16, _grp, 0)

    def _scatter_start(b):
        idxb, rowsb, ssem = bufs[b][3], bufs[b][4], bufs[b][6]
        for j in range(_NSUB):
            pltpu.async_copy(rowsb.at[pl.ds(j * _SUB, _SUB)],
                             acc.at[idxb.at[j]], ssem, add=True)

    def _scatter_wait(b):
        idxb, rowsb, ssem = bufs[b][3], bufs[b][4], bufs[b][6]
        for j in range(_NSUB):
            pltpu.make_async_copy(rowsb.at[pl.ds(j * _SUB, _SUB)],
                                  acc.at[idxb.at[j]], ssem).wait()

    _load_start(jnp.int32(0), 0)
    _load_wait(jnp.int32(0), 0)
    _gather_start(0)
    _load_start(jnp.int32(1), 1)

    def _body(k, carry):
        for i in range(_NBUF):
            b = i
            nb = (i + 1) % _NBUF
            fb = (i + 2) % _NBUF
            m = k * _NBUF + i
            if i == _NBUF - 1:
                _scatter_wait(nb)
            else:
                @pl.when(k > 0)
                def _():
                    _scatter_wait(nb)
            _load_start(m + 2, fb)
            _load_wait(m + 1, nb)
            _gather_start(nb)
            _index(b)
            _gather_wait(b)
            _scale(b)
            _scatter_start(b)
        return carry

    _NLOOP = (_NMACRO - 4) // _NBUF
    lax.fori_loop(0, _NLOOP, _body, 0)

    m0 = _NLOOP * _NBUF
    _scatter_wait(1)
    _load_start(m0 + 2, 2)
    _load_wait(m0 + 1, 1)
    _gather_start(1)
    _index(0)
    _gather_wait(0)
    _scale(0)
    _scatter_start(0)

    _scatter_wait(2)
    _load_last_start(0)
    _load_wait(m0 + 2, 2)
    _gather_start(2)
    _index(1)
    _gather_wait(1)
    _scale(1)
    _scatter_start(1)

    _scatter_wait(0)
    _load_last_wait(0)
    _gather_start(0)
    _index(2)
    _gather_wait(2)
    _scale(2)
    _scatter_start(2)

    _index(0)
    _gather_wait(0)
    _scale(0)
    _scatter_start(0)

    _scatter_wait(1)
    _scatter_wait(2)
    _scatter_wait(0)

    plsc.subcore_barrier()

    start = jnp.minimum(s * _CP_ROWS, _HALF - _CP_ROWS)
    pltpu.sync_copy(acc.at[pl.ds(start, _CP_ROWS)],
                    y_hbm.at[pl.ds(lo + start, _CP_ROWS)])


def _mean_body(a_ref, b_ref, c_ref, o_ref):
    o_ref[...] = (a_ref[...] + b_ref[...] + c_ref[...]) * (1.0 / 3.0)


_R2 = _N * _EMB // 128
_BLK = 1000
_mean3 = pl.pallas_call(
    _mean_body,
    grid=(_R2 // _BLK,),
    in_specs=[pl.BlockSpec((_BLK, 128), lambda i: (i, 0))] * 3,
    out_specs=pl.BlockSpec((_BLK, 128), lambda i: (i, 0)),
    out_shape=jax.ShapeDtypeStruct((_R2, 128), jnp.float32),
)


def kernel(user_emb, item_emb, adj_val, adj_row, adj_col):
    x0 = jnp.concatenate([user_emb, item_emb], axis=0)
    y1 = _spmm(x0, adj_row, adj_col, adj_val)
    y2 = _spmm(y1, adj_row, adj_col, adj_val)
    y3 = _spmm(y2, adj_row, adj_col, adj_val)
    m = _mean3(y1.reshape(_R2, 128), y2.reshape(_R2, 128), y3.reshape(_R2, 128))
    m = m.reshape(_N, _EMB)
    return m[:_USER_NUM], m[_USER_NUM:]
